# baseline (speedup 1.0000x reference)
import functools
import math

import jax
import jax.numpy as jnp
from jax.experimental import pallas as pl
from jax.experimental.pallas import tpu as pltpu

_E = 8
_L2 = 15
_L3 = 32
_AUX_ALPHA = 0.01
_Z_ALPHA = 0.001
_NBUF = 4
_KSPLIT = 4


def _fused_moe_kernel(nb, blk, dk,
                      xe_hbm, xr_hbm, rwt_ref, rb_ref, w1t_ref, b1_ref,
                      w2big_ref, b2_ref, w3sel_ref, b3_ref, hosel_ref,
                      out_ref, frac_ref, avg_ref, scal_ref,
                      xe_buf0, xe_buf1, xe_buf2, xe_buf3,
                      xr_buf, xe_sem0, xe_sem1, xe_sem2, xe_sem3, xr_sem,
                      acc_hard, acc_prob, acc_lse2, acc_ent, acc_maxp):
    xe_bufs = (xe_buf0, xe_buf1, xe_buf2, xe_buf3)
    xe_sems = (xe_sem0, xe_sem1, xe_sem2, xe_sem3)
    i = pl.program_id(0)
    f32 = jnp.float32

    def start_copy(b):
        s = jax.lax.rem(b, _NBUF)
        for j in range(_KSPLIT):
            pltpu.make_async_copy(
                xe_hbm.at[pl.ds(b * blk, blk), pl.ds(j * dk, dk)],
                xe_bufs[j].at[s], xe_sems[j].at[s]
            ).start()
        pltpu.make_async_copy(
            xr_hbm.at[pl.ds(b * blk, blk), :], xr_buf.at[s], xr_sem.at[s]
        ).start()

    @pl.when(i == 0)
    def _prologue():
        for j in range(min(_NBUF, nb)):
            start_copy(j)
        acc_hard[...] = jnp.zeros_like(acc_hard)
        acc_prob[...] = jnp.zeros_like(acc_prob)
        acc_lse2[...] = jnp.zeros_like(acc_lse2)
        acc_ent[...] = jnp.zeros_like(acc_ent)
        acc_maxp[...] = jnp.zeros_like(acc_maxp)

    @pl.when(jnp.logical_and(i > 0, i + _NBUF - 1 < nb))
    def _steady():
        start_copy(i + _NBUF - 1)

    s = jax.lax.rem(i, _NBUF)
    for j in range(_KSPLIT):
        pltpu.make_async_copy(
            xe_hbm.at[pl.ds(i * blk, blk), pl.ds(j * dk, dk)],
            xe_bufs[j].at[s], xe_sems[j].at[s]).wait()
    pltpu.make_async_copy(
        xr_hbm.at[pl.ds(i * blk, blk), :], xr_buf.at[s], xr_sem.at[s]).wait()

    xr = xr_buf[s]
    logits = jnp.dot(xr, rwt_ref[...], preferred_element_type=f32) + rb_ref[...]
    mx = jnp.max(logits, axis=-1, keepdims=True)
    ex = jnp.exp(logits - mx)
    se = jnp.sum(ex, axis=-1, keepdims=True)
    probs = ex / se
    lse = mx + jnp.log(se)
    iota = jax.lax.broadcasted_iota(jnp.int32, logits.shape, 1)
    first_max = jnp.min(jnp.where(logits >= mx, iota, _E), axis=-1, keepdims=True)
    onef = (iota == first_max).astype(f32)

    acc_hard[...] += jnp.sum(onef, axis=0, keepdims=True)
    acc_prob[...] += jnp.sum(probs, axis=0, keepdims=True)
    acc_lse2[...] += jnp.sum(lse * lse, axis=None, keepdims=True)
    plog = jnp.log(jnp.clip(probs, 1e-9, None))
    acc_ent[...] += -jnp.sum(probs * plog, axis=None, keepdims=True)
    acc_maxp[...] += jnp.sum(jnp.max(probs, axis=-1, keepdims=True), axis=None,
                             keepdims=True)

    w1t = w1t_ref[...]
    h = jnp.dot(xe_buf0[s], w1t[0 * dk:1 * dk, :], preferred_element_type=f32)
    for j in range(1, _KSPLIT):
        h += jnp.dot(xe_bufs[j][s], w1t[j * dk:(j + 1) * dk, :],
                     preferred_element_type=f32)
    h += b1_ref[...]
    scale = 255.0 / 256.0
    l1x = jnp.concatenate(
        [jnp.clip(h * h * scale, 0.0, 1.0), jnp.clip(h, 0.0, 1.0)], axis=1)
    l2c = jnp.dot(l1x, w2big_ref[...], preferred_element_type=f32) + b2_ref[...]
    l2x = jnp.clip(l2c, 0.0, 1.0)
    o3 = jnp.dot(l2x, w3sel_ref[...], preferred_element_type=f32) + b3_ref[...]
    ho = jnp.dot(h, hosel_ref[...], preferred_element_type=f32)
    out_ref[...] = jnp.sum(onef * (o3 + ho), axis=1, keepdims=True)

    @pl.when(i == nb - 1)
    def _fin():
        n_tok = float(nb * blk)
        frac = acc_hard[...] / n_tok
        avg = acc_prob[...] / n_tok
        frac_ref[...] = frac
        avg_ref[...] = avg
        aux = _E * jnp.sum(frac * avg, axis=None, keepdims=True)
        z = acc_lse2[...] / n_tok
        ent = acc_ent[...] / (n_tok * math.log(_E))
        top1 = acc_maxp[...] / n_tok
        rl = _AUX_ALPHA * aux + _Z_ALPHA * z
        scal_ref[...] = jnp.concatenate(
            [rl, aux, z, ent, top1, jnp.zeros((1, 3), f32)], axis=1)


def kernel(expert_input, router_input, router_w, router_b, l1_w, l1_b, l1_fw,
           l1_fb, l2_w, l2_b, out_w, out_b):
    B, D_E = expert_input.shape
    D_R = router_input.shape[1]
    T = 512
    nb = B // T
    f32 = jnp.float32
    n1 = _L2 + 1
    F = _E * n1
    G = _E * _L3

    rwt = router_w.T
    rb = router_b.reshape(1, _E)
    w1t = (l1_w + jnp.tile(l1_fw, (_E, 1))).T
    b1m = (l1_b + jnp.tile(l1_fb, (_E,))).reshape(1, F)
    idx = jnp.arange(_E)
    w2r = l2_w.reshape(_E, _L3, 2 * _L2)
    sq = w2r[:, :, :_L2].transpose(0, 2, 1)
    lin = w2r[:, :, _L2:].transpose(0, 2, 1)
    Z4 = jnp.zeros((_E, n1, _E, _L3), f32)
    wsq = Z4.at[idx, :_L2, idx, :].set(sq).reshape(F, G)
    wlin = Z4.at[idx, :_L2, idx, :].set(lin).reshape(F, G)
    w2big = jnp.concatenate([wsq, wlin], axis=0)
    b2f = l2_b.reshape(1, G)
    w3sel = jnp.zeros((_E, _L3, _E), f32).at[idx, :, idx].set(out_w).reshape(G, _E)
    b3f = out_b.reshape(1, _E)
    hosel = jnp.zeros((_E, n1, _E), f32).at[idx, _L2, idx].set(1.0).reshape(F, _E)

    dk = D_E // _KSPLIT
    kern = functools.partial(_fused_moe_kernel, nb, T, dk)
    out, frac, avg, scal = pl.pallas_call(
        kern,
        grid=(nb,),
        in_specs=[
            pl.BlockSpec(memory_space=pltpu.MemorySpace.HBM),
            pl.BlockSpec(memory_space=pltpu.MemorySpace.HBM),
            pl.BlockSpec((D_R, _E), lambda i: (0, 0)),
            pl.BlockSpec((1, _E), lambda i: (0, 0)),
            pl.BlockSpec((D_E, F), lambda i: (0, 0)),
            pl.BlockSpec((1, F), lambda i: (0, 0)),
            pl.BlockSpec((2 * F, G), lambda i: (0, 0)),
            pl.BlockSpec((1, G), lambda i: (0, 0)),
            pl.BlockSpec((G, _E), lambda i: (0, 0)),
            pl.BlockSpec((1, _E), lambda i: (0, 0)),
            pl.BlockSpec((F, _E), lambda i: (0, 0)),
        ],
        out_specs=[
            pl.BlockSpec((T, 1), lambda i: (i, 0)),
            pl.BlockSpec((1, _E), lambda i: (0, 0)),
            pl.BlockSpec((1, _E), lambda i: (0, 0)),
            pl.BlockSpec((1, _E), lambda i: (0, 0)),
        ],
        out_shape=[
            jax.ShapeDtypeStruct((B, 1), f32),
            jax.ShapeDtypeStruct((1, _E), f32),
            jax.ShapeDtypeStruct((1, _E), f32),
            jax.ShapeDtypeStruct((1, _E), f32),
        ],
        scratch_shapes=[
            pltpu.VMEM((_NBUF, T, D_E // _KSPLIT), f32),
            pltpu.VMEM((_NBUF, T, D_E // _KSPLIT), f32),
            pltpu.VMEM((_NBUF, T, D_E // _KSPLIT), f32),
            pltpu.VMEM((_NBUF, T, D_E // _KSPLIT), f32),
            pltpu.VMEM((_NBUF, T, D_R), f32),
            pltpu.SemaphoreType.DMA((_NBUF,)),
            pltpu.SemaphoreType.DMA((_NBUF,)),
            pltpu.SemaphoreType.DMA((_NBUF,)),
            pltpu.SemaphoreType.DMA((_NBUF,)),
            pltpu.SemaphoreType.DMA((_NBUF,)),
            pltpu.VMEM((1, _E), f32),
            pltpu.VMEM((1, _E), f32),
            pltpu.VMEM((1, 1), f32),
            pltpu.VMEM((1, 1), f32),
            pltpu.VMEM((1, 1), f32),
        ],
        compiler_params=pltpu.CompilerParams(
            dimension_semantics=("arbitrary",)),
    )(expert_input, router_input, rwt, rb, w1t, b1m, w2big, b2f, w3sel, b3f,
      hosel)

    return (out, scal[0, 0], scal[0, 1], scal[0, 2], frac[0], avg[0],
            scal[0, 3], scal[0, 4])


# speedup vs baseline: 1.0109x; 1.0109x over previous
import functools
import math

import jax
import jax.numpy as jnp
from jax.experimental import pallas as pl
from jax.experimental.pallas import tpu as pltpu

_E = 8
_L2 = 15
_L3 = 32
_AUX_ALPHA = 0.01
_Z_ALPHA = 0.001
_NBUF = 3


def _fused_moe_kernel(nb, blk, dk,
                      xea_ref, xe_hbm, xr_hbm, rwt_ref, rb_ref, w1t_ref,
                      b1_ref, w2big_ref, b2_ref, w3sel_ref, b3_ref, hosel_ref,
                      out_ref, frac_ref, avg_ref, scal_ref,
                      xe_buf, xr_buf, xe_sem, xr_sem,
                      acc_hard, acc_prob, acc_lse2, acc_ent, acc_maxp):
    i = pl.program_id(0)
    f32 = jnp.float32

    def start_copy(b):
        s = jax.lax.rem(b, _NBUF)
        pltpu.make_async_copy(
            xe_hbm.at[pl.ds(b * blk, blk), pl.ds(dk, dk)],
            xe_buf.at[s], xe_sem.at[s]).start()
        pltpu.make_async_copy(
            xr_hbm.at[pl.ds(b * blk, blk), :], xr_buf.at[s], xr_sem.at[s]
        ).start()

    @pl.when(i == 0)
    def _prologue():
        for j in range(min(_NBUF, nb)):
            start_copy(j)
        acc_hard[...] = jnp.zeros_like(acc_hard)
        acc_prob[...] = jnp.zeros_like(acc_prob)
        acc_lse2[...] = jnp.zeros_like(acc_lse2)
        acc_ent[...] = jnp.zeros_like(acc_ent)
        acc_maxp[...] = jnp.zeros_like(acc_maxp)

    @pl.when(jnp.logical_and(i > 0, i + _NBUF - 1 < nb))
    def _steady():
        start_copy(i + _NBUF - 1)

    s = jax.lax.rem(i, _NBUF)
    pltpu.make_async_copy(
        xe_hbm.at[pl.ds(i * blk, blk), pl.ds(dk, dk)],
        xe_buf.at[s], xe_sem.at[s]).wait()
    pltpu.make_async_copy(
        xr_hbm.at[pl.ds(i * blk, blk), :], xr_buf.at[s], xr_sem.at[s]).wait()

    xr = xr_buf[s]
    logits = jnp.dot(xr, rwt_ref[...], preferred_element_type=f32) + rb_ref[...]
    mx = jnp.max(logits, axis=-1, keepdims=True)
    ex = jnp.exp(logits - mx)
    se = jnp.sum(ex, axis=-1, keepdims=True)
    probs = ex / se
    lse = mx + jnp.log(se)
    iota = jax.lax.broadcasted_iota(jnp.int32, logits.shape, 1)
    first_max = jnp.min(jnp.where(logits >= mx, iota, _E), axis=-1, keepdims=True)
    onef = (iota == first_max).astype(f32)

    acc_hard[...] += jnp.sum(onef, axis=0, keepdims=True)
    acc_prob[...] += jnp.sum(probs, axis=0, keepdims=True)
    acc_lse2[...] += jnp.sum(lse * lse, axis=None, keepdims=True)
    plog = jnp.log(jnp.clip(probs, 1e-9, None))
    acc_ent[...] += -jnp.sum(probs * plog, axis=None, keepdims=True)
    acc_maxp[...] += jnp.sum(jnp.max(probs, axis=-1, keepdims=True), axis=None,
                             keepdims=True)

    w1t = w1t_ref[...]
    h = jnp.dot(xea_ref[...], w1t[0:dk, :], preferred_element_type=f32)
    h += jnp.dot(xe_buf[s], w1t[dk:2 * dk, :], preferred_element_type=f32)
    h += b1_ref[...]
    scale = 255.0 / 256.0
    l1x = jnp.concatenate(
        [jnp.clip(h * h * scale, 0.0, 1.0), jnp.clip(h, 0.0, 1.0)], axis=1)
    l2c = jnp.dot(l1x, w2big_ref[...], preferred_element_type=f32) + b2_ref[...]
    l2x = jnp.clip(l2c, 0.0, 1.0)
    o3 = jnp.dot(l2x, w3sel_ref[...], preferred_element_type=f32) + b3_ref[...]
    ho = jnp.dot(h, hosel_ref[...], preferred_element_type=f32)
    out_ref[...] = jnp.sum(onef * (o3 + ho), axis=1, keepdims=True)

    @pl.when(i == nb - 1)
    def _fin():
        n_tok = float(nb * blk)
        frac = acc_hard[...] / n_tok
        avg = acc_prob[...] / n_tok
        frac_ref[...] = frac
        avg_ref[...] = avg
        aux = _E * jnp.sum(frac * avg, axis=None, keepdims=True)
        z = acc_lse2[...] / n_tok
        ent = acc_ent[...] / (n_tok * math.log(_E))
        top1 = acc_maxp[...] / n_tok
        rl = _AUX_ALPHA * aux + _Z_ALPHA * z
        scal_ref[...] = jnp.concatenate(
            [rl, aux, z, ent, top1, jnp.zeros((1, 3), f32)], axis=1)


def kernel(expert_input, router_input, router_w, router_b, l1_w, l1_b, l1_fw,
           l1_fb, l2_w, l2_b, out_w, out_b):
    B, D_E = expert_input.shape
    D_R = router_input.shape[1]
    T = 512
    nb = B // T
    dk = D_E // 2
    f32 = jnp.float32
    n1 = _L2 + 1
    F = _E * n1
    G = _E * _L3

    rwt = router_w.T
    rb = router_b.reshape(1, _E)
    w1t = (l1_w + jnp.tile(l1_fw, (_E, 1))).T
    b1m = (l1_b + jnp.tile(l1_fb, (_E,))).reshape(1, F)
    idx = jnp.arange(_E)
    w2r = l2_w.reshape(_E, _L3, 2 * _L2)
    sq = w2r[:, :, :_L2].transpose(0, 2, 1)
    lin = w2r[:, :, _L2:].transpose(0, 2, 1)
    Z4 = jnp.zeros((_E, n1, _E, _L3), f32)
    wsq = Z4.at[idx, :_L2, idx, :].set(sq).reshape(F, G)
    wlin = Z4.at[idx, :_L2, idx, :].set(lin).reshape(F, G)
    w2big = jnp.concatenate([wsq, wlin], axis=0)
    b2f = l2_b.reshape(1, G)
    w3sel = jnp.zeros((_E, _L3, _E), f32).at[idx, :, idx].set(out_w).reshape(G, _E)
    b3f = out_b.reshape(1, _E)
    hosel = jnp.zeros((_E, n1, _E), f32).at[idx, _L2, idx].set(1.0).reshape(F, _E)

    kern = functools.partial(_fused_moe_kernel, nb, T, dk)
    out, frac, avg, scal = pl.pallas_call(
        kern,
        grid=(nb,),
        in_specs=[
            pl.BlockSpec((T, dk), lambda i: (i, 0)),
            pl.BlockSpec(memory_space=pltpu.MemorySpace.HBM),
            pl.BlockSpec(memory_space=pltpu.MemorySpace.HBM),
            pl.BlockSpec((D_R, _E), lambda i: (0, 0)),
            pl.BlockSpec((1, _E), lambda i: (0, 0)),
            pl.BlockSpec((D_E, F), lambda i: (0, 0)),
            pl.BlockSpec((1, F), lambda i: (0, 0)),
            pl.BlockSpec((2 * F, G), lambda i: (0, 0)),
            pl.BlockSpec((1, G), lambda i: (0, 0)),
            pl.BlockSpec((G, _E), lambda i: (0, 0)),
            pl.BlockSpec((1, _E), lambda i: (0, 0)),
            pl.BlockSpec((F, _E), lambda i: (0, 0)),
        ],
        out_specs=[
            pl.BlockSpec((T, 1), lambda i: (i, 0)),
            pl.BlockSpec((1, _E), lambda i: (0, 0)),
            pl.BlockSpec((1, _E), lambda i: (0, 0)),
            pl.BlockSpec((1, _E), lambda i: (0, 0)),
        ],
        out_shape=[
            jax.ShapeDtypeStruct((B, 1), f32),
            jax.ShapeDtypeStruct((1, _E), f32),
            jax.ShapeDtypeStruct((1, _E), f32),
            jax.ShapeDtypeStruct((1, _E), f32),
        ],
        scratch_shapes=[
            pltpu.VMEM((_NBUF, T, dk), f32),
            pltpu.VMEM((_NBUF, T, D_R), f32),
            pltpu.SemaphoreType.DMA((_NBUF,)),
            pltpu.SemaphoreType.DMA((_NBUF,)),
            pltpu.VMEM((1, _E), f32),
            pltpu.VMEM((1, _E), f32),
            pltpu.VMEM((1, 1), f32),
            pltpu.VMEM((1, 1), f32),
            pltpu.VMEM((1, 1), f32),
        ],
        compiler_params=pltpu.CompilerParams(
            dimension_semantics=("arbitrary",)),
    )(expert_input, expert_input, router_input, rwt, rb, w1t, b1m, w2big, b2f,
      w3sel, b3f, hosel)

    return (out, scal[0, 0], scal[0, 1], scal[0, 2], frac[0], avg[0],
            scal[0, 3], scal[0, 4])
